# Initial kernel scaffold; baseline (speedup 1.0000x reference)
import jax
import jax.numpy as jnp
from jax.experimental import pallas as pl


def kernel(x, edge_index, graph_ids, add_features, W0, b0, gcr_W, gcr_b, rW1, rb1, rW2, rb2, rW3, rb3):
    raise NotImplementedError("write your pallas kernel here")


# speedup vs baseline: 3.9554x; 3.9554x over previous
import functools

import jax
import jax.numpy as jnp
from jax import lax
from jax.experimental import pallas as pl
from jax.experimental.pallas import tpu as pltpu
from jax.experimental.pallas import tpu_sc as plsc

_N = 10000
_E = 320000
_H = 128
_B = 16
_NW = 32
_EPW = _E // _NW
_CH = 80
_NCH = _EPW // _CH
_NP = 10240
_RPT = _NP // 16

_SC_CACHE = {}


def _sc_kernels():
    if "deg" in _SC_CACHE:
        return _SC_CACHE["deg"], _SC_CACHE["msg"]

    mesh = plsc.VectorSubcoreMesh(core_axis_name="c", subcore_axis_name="s")

    @functools.partial(
        pl.kernel,
        mesh=mesh,
        out_type=jax.ShapeDtypeStruct((2, 2, _NP, _H), jnp.float32),
        scratch_types=[
            pltpu.VMEM((_CH,), jnp.int32),
            pltpu.VMEM((_CH, _H), jnp.float32),
            pltpu.VMEM_SHARED((_NP, _H), jnp.float32),
        ],
    )
    def deg_kernel(src_hbm, dst_hbm, ones_hbm, zero_hbm, out_hbm,
                   idxv, ones_v, acc):
        cid = lax.axis_index("c")
        sid = lax.axis_index("s")
        pltpu.sync_copy(ones_hbm, ones_v)
        base = (cid * 16 + sid) * _EPW
        my_rows = pl.ds(sid * _RPT, _RPT)

        for which, e_hbm in ((0, src_hbm), (1, dst_hbm)):
            pltpu.sync_copy(zero_hbm, acc.at[my_rows])
            plsc.subcore_barrier()

            def body(g, carry):
                pltpu.sync_copy(e_hbm.at[pl.ds(base + g * _CH, _CH)], idxv)
                pltpu.sync_copy(ones_v, acc.at[idxv], add=True)
                return carry

            lax.fori_loop(0, _NCH, body, 0)
            plsc.subcore_barrier()
            pltpu.sync_copy(acc.at[my_rows],
                            out_hbm.at[cid, which, my_rows])

    @functools.partial(
        pl.kernel,
        mesh=mesh,
        out_type=jax.ShapeDtypeStruct((2, _NP, _H), jnp.float32),
        scratch_types=[
            pltpu.VMEM((_CH,), jnp.int32),
            pltpu.VMEM((_CH,), jnp.int32),
            pltpu.VMEM((_CH, _H), jnp.float32),
            pltpu.VMEM_SHARED((_NP, _H), jnp.float32),
            pltpu.SemaphoreType.DMA,
        ],
    )
    def msg_kernel(h_hbm, src_hbm, dst_hbm, zero_hbm, out_hbm, srcv, dstv,
                   rows, acc, sem):
        cid = lax.axis_index("c")
        sid = lax.axis_index("s")
        pltpu.sync_copy(zero_hbm, acc.at[pl.ds(sid * _RPT, _RPT)])
        plsc.subcore_barrier()

        base = (cid * 16 + sid) * _EPW

        def body(g, carry):
            off = base + g * _CH
            pltpu.sync_copy(src_hbm.at[pl.ds(off, _CH)], srcv)
            pltpu.sync_copy(dst_hbm.at[pl.ds(off, _CH)], dstv)
            pltpu.async_copy(h_hbm.at[srcv], rows, sem).wait()
            pltpu.sync_copy(rows, acc.at[dstv], add=True)
            return carry

        lax.fori_loop(0, _NCH, body, 0)
        plsc.subcore_barrier()
        pltpu.sync_copy(acc.at[pl.ds(sid * _RPT, _RPT)],
                        out_hbm.at[cid, pl.ds(sid * _RPT, _RPT)])

    _SC_CACHE["deg"] = deg_kernel
    _SC_CACHE["msg"] = msg_kernel
    return deg_kernel, msg_kernel


def _round0_body(x_ref, W_ref, dgo_ref, dgi_ref, h_ref, dO_ref, dI_ref):
    d_out = jnp.sum(dgo_ref[...], axis=0)
    d_in = jnp.sum(dgi_ref[...], axis=0)
    dO = lax.rsqrt(jnp.maximum(d_out, 1.0)).reshape(_N, 1)
    dI = lax.rsqrt(jnp.maximum(d_in, 1.0)).reshape(_N, 1)
    dO_ref[...] = dO
    dI_ref[...] = dI
    h_ref[...] = jnp.dot(x_ref[...] * dO, W_ref[...],
                         preferred_element_type=jnp.float32)


def _round_body(relu, parts_ref, dI_ref, dO_ref, b_ref, W_ref, h_ref):
    f = parts_ref[0, :_N, :] + parts_ref[1, :_N, :]
    f = f * dI_ref[...] + b_ref[...]
    if relu:
        f = jnp.maximum(f, 0.0)
    h_ref[...] = jnp.dot(f * dO_ref[...], W_ref[...],
                         preferred_element_type=jnp.float32)


def _epilogue_body(parts_ref, dI_ref, b_ref, gid_ref, add_ref,
                   w1a_ref, w1b_ref, b1_ref, w2_ref, b2_ref, w3_ref, b3_ref,
                   out_ref):
    f = parts_ref[0, :_N, :] + parts_ref[1, :_N, :]
    f = jnp.maximum(f * dI_ref[...] + b_ref[...], 0.0)
    iota = lax.broadcasted_iota(jnp.int32, (_B, 1), 0)
    onehot = (gid_ref[...] == iota).astype(jnp.float32)
    sums = jnp.dot(onehot, f, preferred_element_type=jnp.float32)
    counts = jnp.maximum(jnp.sum(onehot, axis=1), 1.0).reshape(_B, 1)
    mean = sums / counts
    h1 = (jnp.dot(mean, w1a_ref[...], preferred_element_type=jnp.float32)
          + jnp.dot(add_ref[...], w1b_ref[...], preferred_element_type=jnp.float32)
          + b1_ref[...])
    h1 = jnp.where(h1 > 0, h1, 0.01 * h1)
    h2 = jnp.dot(h1, w2_ref[...], preferred_element_type=jnp.float32) + b2_ref[...]
    h2 = jnp.where(h2 > 0, h2, 0.01 * h2)
    out_ref[...] = jnp.dot(h2, w3_ref[...],
                           preferred_element_type=jnp.float32) + b3_ref[...]


_f32 = jnp.float32

_round0_call = pl.pallas_call(
    _round0_body,
    out_shape=[jax.ShapeDtypeStruct((_N, _H), _f32),
               jax.ShapeDtypeStruct((_N, 1), _f32),
               jax.ShapeDtypeStruct((_N, 1), _f32)],
)
_round_relu_call = pl.pallas_call(
    functools.partial(_round_body, True),
    out_shape=jax.ShapeDtypeStruct((_N, _H), _f32),
)
_round_norelu_call = pl.pallas_call(
    functools.partial(_round_body, False),
    out_shape=jax.ShapeDtypeStruct((_N, _H), _f32),
)
_epilogue_call = pl.pallas_call(
    _epilogue_body,
    out_shape=jax.ShapeDtypeStruct((_B, 1), _f32),
)


def kernel(x, edge_index, graph_ids, add_features, W0, b0, gcr_W, gcr_b,
           rW1, rb1, rW2, rb2, rW3, rb3):
    src = edge_index[0]
    dst = edge_index[1]
    _deg_kernel, _msg_kernel = _sc_kernels()

    ones_rows = jnp.ones((_CH, _H), _f32)
    zeros = jnp.zeros((_RPT, _H), _f32)
    degp = _deg_kernel(src, dst, ones_rows, zeros)
    dgo = degp[:, 0, :_N, 0]
    dgi = degp[:, 1, :_N, 0]

    h, dO, dI = _round0_call(x, W0, dgo, dgi)

    parts = _msg_kernel(h, src, dst, zeros)
    for k in range(1, 11):
        i, j = divmod(k - 1, 2)
        W = gcr_W[i, j]
        b_prev = b0.reshape(1, _H) if k == 1 else gcr_b[(k - 2) // 2, (k - 2) % 2].reshape(1, _H)
        call = _round_norelu_call if k == 1 else _round_relu_call
        h = call(parts, dI, dO, b_prev, W)
        parts = _msg_kernel(h, src, dst, zeros)

    b_last = gcr_b[4, 1].reshape(1, _H)
    gid_row = graph_ids.reshape(1, _N)
    out = _epilogue_call(parts, dI, b_last, gid_row, add_features,
                         rW1[:_H], rW1[_H:], rb1.reshape(1, 1024),
                         rW2, rb2.reshape(1, 512), rW3, rb3.reshape(1, 1))
    return out[:, 0]


# speedup vs baseline: 8.6393x; 2.1842x over previous
import functools

import jax
import jax.numpy as jnp
from jax import lax
from jax.experimental import pallas as pl
from jax.experimental.pallas import tpu as pltpu
from jax.experimental.pallas import tpu_sc as plsc

_N = 10000
_E = 320000
_H = 128
_B = 16
_NW = 32
_EPW = _E // _NW
_CH = 80
_NCH = _EPW // _CH
_NP = 10240
_RPT = _NP // 16

_SC_CACHE = {}


def _sc_kernels():
    if "deg" in _SC_CACHE:
        return _SC_CACHE["deg"], _SC_CACHE["msg"]

    mesh = plsc.VectorSubcoreMesh(core_axis_name="c", subcore_axis_name="s")

    @functools.partial(
        pl.kernel,
        mesh=mesh,
        out_type=jax.ShapeDtypeStruct((2, 2, _NP, _H), jnp.float32),
        scratch_types=[
            pltpu.VMEM((_CH,), jnp.int32),
            pltpu.VMEM((_CH, _H), jnp.float32),
            pltpu.VMEM_SHARED((_NP, _H), jnp.float32),
        ],
    )
    def deg_kernel(src_hbm, dst_hbm, ones_hbm, zero_hbm, out_hbm,
                   idxv, ones_v, acc):
        cid = lax.axis_index("c")
        sid = lax.axis_index("s")
        pltpu.sync_copy(ones_hbm, ones_v)
        base = (cid * 16 + sid) * _EPW
        my_rows = pl.ds(sid * _RPT, _RPT)

        for which, e_hbm in ((0, src_hbm), (1, dst_hbm)):
            pltpu.sync_copy(zero_hbm, acc.at[my_rows])
            plsc.subcore_barrier()

            def body(g, carry):
                pltpu.sync_copy(e_hbm.at[pl.ds(base + g * _CH, _CH)], idxv)
                pltpu.sync_copy(ones_v, acc.at[idxv], add=True)
                return carry

            lax.fori_loop(0, _NCH, body, 0)
            plsc.subcore_barrier()
            pltpu.sync_copy(acc.at[my_rows],
                            out_hbm.at[cid, which, my_rows])

    @functools.partial(
        pl.kernel,
        mesh=mesh,
        out_type=jax.ShapeDtypeStruct((2, _NP, _H), jnp.float32),
        scratch_types=[
            pltpu.VMEM((_EPW,), jnp.int32),
            pltpu.VMEM((_NCH, _CH), jnp.int32),
            pltpu.VMEM((_CH, _H), jnp.float32),
            pltpu.VMEM((_CH, _H), jnp.float32),
            pltpu.VMEM_SHARED((_NP, _H), jnp.float32),
            pltpu.SemaphoreType.DMA,
            pltpu.SemaphoreType.DMA,
        ],
    )
    def msg_kernel(h_hbm, src_hbm, dst_hbm, zero_hbm, out_hbm, srcv, dstm,
                   rows0, rows1, acc, sem0, sem1):
        cid = lax.axis_index("c")
        sid = lax.axis_index("s")
        wid = cid * 16 + sid
        pltpu.sync_copy(src_hbm.at[pl.ds(wid * _EPW, _EPW)], srcv)
        pltpu.sync_copy(dst_hbm.at[wid], dstm)
        pltpu.sync_copy(zero_hbm, acc.at[pl.ds(sid * _RPT, _RPT)])
        plsc.subcore_barrier()

        def gather(g, rows, sem):
            return pltpu.async_copy(h_hbm.at[srcv.at[pl.ds(g * _CH, _CH)]],
                                    rows, sem)

        def gwait(g, rows, sem):
            pltpu.make_async_copy(h_hbm.at[srcv.at[pl.ds(g * _CH, _CH)]],
                                  rows, sem).wait()

        def scat(g, rows):
            pltpu.sync_copy(rows, acc.at[dstm.at[g]], add=True)

        gather(0, rows0, sem0)

        def pair(t, carry):
            g0 = 2 * t
            gather(g0 + 1, rows1, sem1)
            gwait(g0, rows0, sem0)
            scat(g0, rows0)
            gather(g0 + 2, rows0, sem0)
            gwait(g0 + 1, rows1, sem1)
            scat(g0 + 1, rows1)
            return carry

        lax.fori_loop(0, (_NCH - 1) // 2, pair, 0)
        gwait(_NCH - 1, rows0, sem0)
        scat(_NCH - 1, rows0)

        plsc.subcore_barrier()
        pltpu.sync_copy(acc.at[pl.ds(sid * _RPT, _RPT)],
                        out_hbm.at[cid, pl.ds(sid * _RPT, _RPT)])

    _SC_CACHE["deg"] = deg_kernel
    _SC_CACHE["msg"] = msg_kernel
    return deg_kernel, msg_kernel


def _round0_body(x_ref, W_ref, dgo_ref, dgi_ref, h_ref, dO_ref, dI_ref):
    d_out = jnp.sum(dgo_ref[...], axis=0)
    d_in = jnp.sum(dgi_ref[...], axis=0)
    dO = lax.rsqrt(jnp.maximum(d_out, 1.0)).reshape(_N, 1)
    dI = lax.rsqrt(jnp.maximum(d_in, 1.0)).reshape(_N, 1)
    dO_ref[...] = dO
    dI_ref[...] = dI
    h_ref[...] = jnp.dot(x_ref[...] * dO, W_ref[...],
                         preferred_element_type=jnp.float32)


def _round_body(relu, parts_ref, dI_ref, dO_ref, b_ref, W_ref, h_ref):
    f = parts_ref[0, :_N, :] + parts_ref[1, :_N, :]
    f = f * dI_ref[...] + b_ref[...]
    if relu:
        f = jnp.maximum(f, 0.0)
    h_ref[...] = jnp.dot(f * dO_ref[...], W_ref[...],
                         preferred_element_type=jnp.float32)


def _epilogue_body(parts_ref, dI_ref, b_ref, gid_ref, add_ref,
                   w1a_ref, w1b_ref, b1_ref, w2_ref, b2_ref, w3_ref, b3_ref,
                   out_ref):
    f = parts_ref[0, :_N, :] + parts_ref[1, :_N, :]
    f = jnp.maximum(f * dI_ref[...] + b_ref[...], 0.0)
    iota = lax.broadcasted_iota(jnp.int32, (_B, 1), 0)
    onehot = (gid_ref[...] == iota).astype(jnp.float32)
    sums = jnp.dot(onehot, f, preferred_element_type=jnp.float32)
    counts = jnp.maximum(jnp.sum(onehot, axis=1), 1.0).reshape(_B, 1)
    mean = sums / counts
    h1 = (jnp.dot(mean, w1a_ref[...], preferred_element_type=jnp.float32)
          + jnp.dot(add_ref[...], w1b_ref[...], preferred_element_type=jnp.float32)
          + b1_ref[...])
    h1 = jnp.where(h1 > 0, h1, 0.01 * h1)
    h2 = jnp.dot(h1, w2_ref[...], preferred_element_type=jnp.float32) + b2_ref[...]
    h2 = jnp.where(h2 > 0, h2, 0.01 * h2)
    out_ref[...] = jnp.dot(h2, w3_ref[...],
                           preferred_element_type=jnp.float32) + b3_ref[...]


_f32 = jnp.float32

_round0_call = pl.pallas_call(
    _round0_body,
    out_shape=[jax.ShapeDtypeStruct((_N, _H), _f32),
               jax.ShapeDtypeStruct((_N, 1), _f32),
               jax.ShapeDtypeStruct((_N, 1), _f32)],
)
_round_relu_call = pl.pallas_call(
    functools.partial(_round_body, True),
    out_shape=jax.ShapeDtypeStruct((_N, _H), _f32),
)
_round_norelu_call = pl.pallas_call(
    functools.partial(_round_body, False),
    out_shape=jax.ShapeDtypeStruct((_N, _H), _f32),
)
_epilogue_call = pl.pallas_call(
    _epilogue_body,
    out_shape=jax.ShapeDtypeStruct((_B, 1), _f32),
)


def kernel(x, edge_index, graph_ids, add_features, W0, b0, gcr_W, gcr_b,
           rW1, rb1, rW2, rb2, rW3, rb3):
    src = edge_index[0]
    dst = edge_index[1]
    _deg_kernel, _msg_kernel = _sc_kernels()

    ones_rows = jnp.ones((_CH, _H), _f32)
    zeros = jnp.zeros((_RPT, _H), _f32)
    degp = _deg_kernel(src, dst, ones_rows, zeros)
    dgo = degp[:, 0, :_N, 0]
    dgi = degp[:, 1, :_N, 0]

    h, dO, dI = _round0_call(x, W0, dgo, dgi)
    dst3 = dst.reshape(_NW, _NCH, _CH)

    parts = _msg_kernel(h, src, dst3, zeros)
    for k in range(1, 11):
        i, j = divmod(k - 1, 2)
        W = gcr_W[i, j]
        b_prev = b0.reshape(1, _H) if k == 1 else gcr_b[(k - 2) // 2, (k - 2) % 2].reshape(1, _H)
        call = _round_norelu_call if k == 1 else _round_relu_call
        h = call(parts, dI, dO, b_prev, W)
        parts = _msg_kernel(h, src, dst3, zeros)

    b_last = gcr_b[4, 1].reshape(1, _H)
    gid_row = graph_ids.reshape(1, _N)
    out = _epilogue_call(parts, dI, b_last, gid_row, add_features,
                         rW1[:_H], rW1[_H:], rb1.reshape(1, 1024),
                         rW2, rb2.reshape(1, 512), rW3, rb3.reshape(1, 1))
    return out[:, 0]
